# Initial kernel scaffold; baseline (speedup 1.0000x reference)
import jax
import jax.numpy as jnp
from jax.experimental import pallas as pl


def kernel(edge_attr, table, W1, b1, W2, b2):
    raise NotImplementedError("write your pallas kernel here")


# speedup vs baseline: 3.2515x; 3.2515x over previous
import functools

import jax
import jax.numpy as jnp
from jax.experimental import pallas as pl
from jax.experimental.pallas import tpu as pltpu
from jax.experimental.pallas import tpu_sc as plsc

_GATHER_WINDOW = 128


def _sc_gather(table, ids):
    num = ids.shape[0]
    depth = table.shape[1]
    ids2 = ids.reshape(1, num)
    mesh = plsc.VectorSubcoreMesh(core_axis_name="c", subcore_axis_name="s")

    @functools.partial(
        pl.kernel,
        out_type=jax.ShapeDtypeStruct((num, depth), table.dtype),
        mesh=mesh,
        compiler_params=pltpu.CompilerParams(use_tc_tiling_on_sc=False),
    )
    def gather_kernel(table_hbm, ids_hbm, out_hbm):
        def body(i_vmem, o_vmem):
            pltpu.sync_copy(table_hbm.at[i_vmem.at[0]], o_vmem)

        pltpu.emit_pipeline(
            body,
            grid=(num // _GATHER_WINDOW,),
            in_specs=[
                pl.BlockSpec((1, _GATHER_WINDOW), lambda i: (0, i)),
            ],
            out_specs=[
                pl.BlockSpec((_GATHER_WINDOW, depth), lambda i: (i, 0)),
            ],
            core_axis_name=("c", "s"),
            dimension_semantics=(pltpu.PARALLEL,),
        )(ids_hbm, out_hbm)

    return gather_kernel(table, ids2)


def _mlp(edge_attr, emb, w1e, w1n, b1r, W2, b2r, block_rows):
    num, feat = edge_attr.shape
    depth = emb.shape[1]
    hid = w1e.shape[1]

    def body(ea_ref, emb_ref, w1e_ref, w1n_ref, b1_ref, w2_ref, b2_ref, o_ref):
        h = jnp.dot(emb_ref[...], w1e_ref[...], preferred_element_type=jnp.float32)
        h = h + jnp.dot(ea_ref[...], w1n_ref[...], preferred_element_type=jnp.float32)
        h = jnp.maximum(h + b1_ref[...], 0.0)
        o_ref[...] = (
            jnp.dot(h, w2_ref[...], preferred_element_type=jnp.float32) + b2_ref[...]
        )

    return pl.pallas_call(
        body,
        grid=(num // block_rows,),
        in_specs=[
            pl.BlockSpec((block_rows, feat), lambda i: (i, 0)),
            pl.BlockSpec((block_rows, depth), lambda i: (i, 0)),
            pl.BlockSpec((depth, hid), lambda i: (0, 0)),
            pl.BlockSpec((feat, hid), lambda i: (0, 0)),
            pl.BlockSpec((1, hid), lambda i: (0, 0)),
            pl.BlockSpec((hid, hid), lambda i: (0, 0)),
            pl.BlockSpec((1, hid), lambda i: (0, 0)),
        ],
        out_specs=pl.BlockSpec((block_rows, hid), lambda i: (i, 0)),
        out_shape=jax.ShapeDtypeStruct((num, hid), jnp.float32),
    )(edge_attr, emb, w1e, w1n, b1r, W2, b2r)


def kernel(edge_attr, table, W1, b1, W2, b2):
    depth = table.shape[1]
    hid = W1.shape[1]
    ids = edge_attr[:, 0].astype(jnp.int32)
    emb = _sc_gather(table, ids)
    w1e = W1[:depth]
    w1n = jnp.concatenate([jnp.zeros((1, hid), W1.dtype), W1[depth:]], axis=0)
    b1r = b1.reshape(1, hid)
    b2r = b2.reshape(1, hid)
    return _mlp(edge_attr, emb, w1e, w1n, b1r, W2, b2r, block_rows=16000)


# speedup vs baseline: 5.5280x; 1.7002x over previous
import functools

import jax
import jax.numpy as jnp
from jax.experimental import pallas as pl
from jax.experimental.pallas import tpu as pltpu
from jax.experimental.pallas import tpu_sc as plsc

_GATHER_WINDOW = 128


def _sc_gather(table, ids):
    num = ids.shape[0]
    depth = table.shape[1]
    ids2 = ids.reshape(1, num)
    mesh = plsc.VectorSubcoreMesh(core_axis_name="c", subcore_axis_name="s")

    @functools.partial(
        pl.kernel,
        out_type=jax.ShapeDtypeStruct((num, depth), table.dtype),
        mesh=mesh,
        compiler_params=pltpu.CompilerParams(use_tc_tiling_on_sc=False),
    )
    def gather_kernel(table_hbm, ids_hbm, out_hbm):
        def body(i_vmem, o_vmem):
            pltpu.sync_copy(table_hbm.at[i_vmem.at[0]], o_vmem)

        pltpu.emit_pipeline(
            body,
            grid=(num // _GATHER_WINDOW,),
            in_specs=[
                pl.BlockSpec((1, _GATHER_WINDOW), lambda i: (0, i)),
            ],
            out_specs=[
                pl.BlockSpec((_GATHER_WINDOW, depth), lambda i: (i, 0)),
            ],
            core_axis_name=("c", "s"),
            dimension_semantics=(pltpu.PARALLEL,),
        )(ids_hbm, out_hbm)

    return gather_kernel(table, ids2)


_PACK = 8


def _mlp_packed(xa, xe, w1a_big, w1e_big, b1_big, w2_big, b2_big, block_rows):
    rows = xa.shape[0]

    def body(xa_ref, xe_ref, w1a_ref, w1e_ref, b1_ref, w2_ref, b2_ref, o_ref):
        h = jnp.dot(xe_ref[...], w1e_ref[...], preferred_element_type=jnp.float32)
        h = h + jnp.dot(xa_ref[...], w1a_ref[...], preferred_element_type=jnp.float32)
        h = jnp.maximum(h + b1_ref[...], 0.0)
        o_ref[...] = (
            jnp.dot(h, w2_ref[...], preferred_element_type=jnp.float32) + b2_ref[...]
        )

    return pl.pallas_call(
        body,
        grid=(rows // block_rows,),
        in_specs=[
            pl.BlockSpec((block_rows, 128), lambda i: (i, 0)),
            pl.BlockSpec((block_rows, 128), lambda i: (i, 0)),
            pl.BlockSpec((128, 256), lambda i: (0, 0)),
            pl.BlockSpec((128, 256), lambda i: (0, 0)),
            pl.BlockSpec((1, 256), lambda i: (0, 0)),
            pl.BlockSpec((256, 256), lambda i: (0, 0)),
            pl.BlockSpec((1, 256), lambda i: (0, 0)),
        ],
        out_specs=pl.BlockSpec((block_rows, 256), lambda i: (i, 0)),
        out_shape=jax.ShapeDtypeStruct((rows, 256), jnp.float32),
    )(xa, xe, w1a_big, w1e_big, b1_big, w2_big, b2_big)


def kernel(edge_attr, table, W1, b1, W2, b2):
    num = edge_attr.shape[0]
    depth = table.shape[1]
    hid = W1.shape[1]
    ids = edge_attr[:, 0].astype(jnp.int32)
    emb = _sc_gather(table, ids)
    eye = jnp.eye(_PACK, dtype=W1.dtype)
    w1e_big = jnp.kron(eye, W1[:depth])
    w1n = jnp.concatenate([jnp.zeros((1, hid), W1.dtype), W1[depth:]], axis=0)
    w1a_big = jnp.kron(eye, w1n)
    w2_big = jnp.kron(eye, W2)
    b1_big = jnp.tile(b1, _PACK).reshape(1, _PACK * hid)
    b2_big = jnp.tile(b2, _PACK).reshape(1, _PACK * hid)
    xa = edge_attr.reshape(num // _PACK, _PACK * depth)
    xe = emb.reshape(num // _PACK, _PACK * depth)
    out = _mlp_packed(xa, xe, w1a_big, w1e_big, b1_big, w2_big, b2_big,
                      block_rows=2000)
    return out.reshape(num, hid)
